# baseline (speedup 1.0000x reference)
import functools

import jax
import jax.numpy as jnp
from jax import lax
from jax.experimental import pallas as pl
from jax.experimental.pallas import tpu as pltpu
from jax.experimental.pallas import tpu_sc as plsc

B, L, D = 4096, 200, 32
N = B * L
NC, NS = 2, 16
NW = NC * NS
PER_W = N // NW
CHUNK = 128
NCHUNK = PER_W // CHUNK
GROUP = 10
NGROUP = NCHUNK // GROUP
NPAIR = NGROUP // 2
GROUP_ROWS = GROUP * CHUNK

_mesh = plsc.VectorSubcoreMesh(core_axis_name="c", subcore_axis_name="s")


@functools.partial(
    pl.kernel,
    mesh=_mesh,
    out_type=jax.ShapeDtypeStruct((N, D), jnp.float32),
    scratch_types=[
        pltpu.VMEM((NCHUNK, CHUNK), jnp.int32),
        pltpu.VMEM((GROUP_ROWS, D), jnp.float32),
        pltpu.VMEM((GROUP_ROWS, D), jnp.float32),
        pltpu.SemaphoreType.DMA,
        pltpu.SemaphoreType.DMA,
        pltpu.SemaphoreType.DMA,
        pltpu.SemaphoreType.DMA,
    ],
    compiler_params=pltpu.CompilerParams(use_tc_tiling_on_sc=False),
)
def _gather_kernel(table_hbm, idx_hbm, out_hbm, idx_v, buf0, buf1,
                   sem_g0, sem_g1, sem_w0, sem_w1):
    wid = lax.axis_index("s") * NC + lax.axis_index("c")
    base = wid * PER_W
    pltpu.sync_copy(idx_hbm.at[pl.ds(wid * NCHUNK, NCHUNK)], idx_v)

    def fire_group(g, buf, sem):
        for j in range(GROUP):
            pltpu.async_copy(
                table_hbm.at[idx_v.at[g * GROUP + j]],
                buf.at[pl.ds(j * CHUNK, CHUNK)],
                sem,
            )

    fire_group(0, buf0, sem_g0)

    def body(i, carry):
        g0 = 2 * i
        fire_group(g0 + 1, buf1, sem_g1)
        pltpu.make_async_copy(
            out_hbm.at[pl.ds(0, GROUP_ROWS)], buf0, sem_g0).wait()
        w0 = pltpu.async_copy(
            buf0, out_hbm.at[pl.ds(base + g0 * GROUP_ROWS, GROUP_ROWS)],
            sem_w0)
        pltpu.make_async_copy(
            out_hbm.at[pl.ds(0, GROUP_ROWS)], buf1, sem_g1).wait()
        w1 = pltpu.async_copy(
            buf1, out_hbm.at[pl.ds(base + (g0 + 1) * GROUP_ROWS, GROUP_ROWS)],
            sem_w1)
        w0.wait()

        @pl.when(i < NPAIR - 1)
        def _():
            fire_group(g0 + 2, buf0, sem_g0)

        w1.wait()
        return carry

    lax.fori_loop(0, NPAIR, body, 0)


def kernel(input_seqs, table):
    idx = input_seqs.reshape(N // CHUNK, CHUNK)
    out = _gather_kernel(table, idx)
    return out.reshape(B, L, D)
